# baseline (speedup 1.0000x reference)
import dataclasses
import functools

import jax
import jax.numpy as jnp
from jax import lax
from jax.experimental import pallas as pl
from jax.experimental.pallas import tpu as pltpu
from jax.experimental.pallas import tpu_sc as plsc

_N = 10000
_E = 320000
_H = 128
_NC = 2
_NS = 16
_TILES = _NC * _NS
_EPT = _E // _TILES
_C = 80
_HC = _C // 2
_CHUNKS = _EPT // _C
_NHALF = 2 * _CHUNKS
_RPT = 624
_WB = 208
_TAIL = _N - _RPT * _NS


def _tc_in_body(x_ref, w_ref, b_ref, a_ref, v_ref, h_ref, s_ref):
    h = jnp.dot(x_ref[...], w_ref[...], preferred_element_type=jnp.float32) + b_ref[...]
    h = jnp.maximum(h, 0.0)
    h_ref[...] = h
    s_ref[...] = jnp.dot(h, a_ref[...], preferred_element_type=jnp.float32) + v_ref[...]


def _tc_combine_body(h_ref, p0_ref, p1_ref, e_ref, a_ref, v_ref,
                     h2_ref, s_ref):
    e = e_ref[...]
    agg = p0_ref[...] + p1_ref[...]
    h2 = jnp.maximum(e * h_ref[...] + (1.0 - e) * agg, 0.0)
    h2_ref[...] = h2
    s_ref[...] = jnp.dot(h2, a_ref[...], preferred_element_type=jnp.float32) + v_ref[...]


def _tc_final_body(h_ref, p0_ref, p1_ref, e_ref, w_ref, b_ref, o_ref):
    e = e_ref[...]
    agg = p0_ref[...] + p1_ref[...]
    h2 = jnp.maximum(e * h_ref[...] + (1.0 - e) * agg, 0.0)
    o_ref[...] = jnp.dot(h2, w_ref[...], preferred_element_type=jnp.float32) + b_ref[...]


def _tc_in(x, w, b, a, v):
    return pl.pallas_call(
        _tc_in_body,
        out_shape=(jax.ShapeDtypeStruct((_N, _H), jnp.float32),
                   jax.ShapeDtypeStruct((_N, _H), jnp.float32)),
    )(x, w, b, a, v)


def _tc_combine(h, p0, p1, e, a, v):
    return pl.pallas_call(
        _tc_combine_body,
        out_shape=(jax.ShapeDtypeStruct((_N, _H), jnp.float32),
                   jax.ShapeDtypeStruct((_N, _H), jnp.float32)),
    )(h, p0, p1, e, a, v)


def _tc_final(h, p0, p1, e, w, b):
    return pl.pallas_call(
        _tc_final_body,
        out_shape=jax.ShapeDtypeStruct((_N, b.shape[-1]), jnp.float32),
    )(h, p0, p1, e, w, b)


def _sc_edge_body(h_hbm, rc_hbm, s1_hbm, s2_hbm, out_hbm,
                  s1_v, s2_v, idx_ring, alpha_v, rows_ring,
                  acc_sh, isems, gsems, ssems):
    c = lax.axis_index("c")
    s = lax.axis_index("s")

    pltpu.sync_copy(s1_hbm, s1_v)
    pltpu.sync_copy(s2_hbm, s2_v)

    @pl.loop(0, _HC)
    def _zero_rows(i):
        for j in range(_H // 16):
            rows_ring[0, i, pl.ds(j * 16, 16)] = jnp.zeros((16,), jnp.float32)

    for k in range(15):
        pltpu.sync_copy(rows_ring.at[0],
                        acc_sh.at[pl.ds(s * _RPT + k * _HC, _HC)])
    pltpu.sync_copy(rows_ring.at[0, pl.ds(0, 24)],
                    acc_sh.at[pl.ds(s * _RPT + 600, 24)])

    @pl.when(s == _NS - 1)
    def _zero_tail():
        pltpu.sync_copy(rows_ring.at[0, pl.ds(0, _TAIL)],
                        acc_sh.at[pl.ds(_RPT * _NS, _TAIL)])

    plsc.subcore_barrier()

    cbase = (c * _NS + s) * _CHUNKS

    def idx_start(k):
        q = lax.rem(k, 4)
        pltpu.async_copy(rc_hbm.at[cbase + k], idx_ring.at[q], isems.at[q])

    def idx_wait(k):
        q = lax.rem(k, 4)
        pltpu.make_async_copy(rc_hbm.at[cbase], idx_ring.at[q],
                              isems.at[q]).wait()

    def _slots(j):
        q = lax.rem(lax.div(j, 2), 4)
        jh = lax.rem(j, 2)
        r = lax.rem(j, 4)
        return q, jh, r

    def gather_start(j):
        q, jh, r = _slots(j)
        pltpu.async_copy(h_hbm.at[idx_ring.at[q, 0, jh]], rows_ring.at[r],
                         gsems.at[r])

    def gather_wait(j):
        q, jh, r = _slots(j)
        pltpu.make_async_copy(h_hbm.at[idx_ring.at[q, 0, jh]],
                              rows_ring.at[r], gsems.at[r]).wait()

    def scatter_start(j):
        q, jh, r = _slots(j)
        pltpu.async_copy(rows_ring.at[r], acc_sh.at[idx_ring.at[q, 1, jh]],
                         ssems.at[r], add=True)

    def scatter_wait(j):
        q, jh, r = _slots(j)
        pltpu.make_async_copy(rows_ring.at[r],
                              acc_sh.at[idx_ring.at[q, 1, jh]],
                              ssems.at[r]).wait()

    def compute(j):
        q, jh, r = _slots(j)
        irf = idx_ring.at[q, 0, jh]
        icf = idx_ring.at[q, 1, jh]
        rv = rows_ring.at[r]

        for off in (0, 16, 24):
            ir = irf[pl.ds(off, 16)]
            ic = icf[pl.ds(off, 16)]
            z = plsc.load_gather(s1_v, [ir]) + plsc.load_gather(s2_v, [ic])
            t = jnp.exp(-2.0 * jnp.abs(z))
            m = (1.0 - t) / (1.0 + t)
            alpha_v[pl.ds(off, 16)] = jnp.sign(z) * m

        @plsc.parallel_loop(0, _HC, unroll=4)
        def _scale(e):
            av = plsc.load_gather(alpha_v, [jnp.broadcast_to(e, (16,))])
            for jj in range(_H // 16):
                sl = pl.ds(jj * 16, 16)
                rv[e, sl] = rv[e, sl] * av

    idx_start(0)
    idx_start(1)
    idx_start(2)
    idx_wait(0)
    gather_start(0)
    gather_start(1)

    @pl.loop(0, _NHALF)
    def _main(j):
        gather_wait(j)

        @pl.when(j >= 2)
        def _drain_scatter():
            scatter_wait(j - 2)

        @pl.when(j + 2 < _NHALF)
        def _next_gather():
            @pl.when(lax.rem(j, 2) == 0)
            def _next_idx_wait():
                idx_wait(lax.div(j + 2, 2))

            gather_start(j + 2)

        @pl.when((lax.rem(j, 2) == 1) & (lax.div(j, 2) + 3 < _CHUNKS))
        def _prefetch_idx():
            idx_start(lax.div(j, 2) + 3)

        compute(j)
        scatter_start(j)

    scatter_wait(_NHALF - 2)
    scatter_wait(_NHALF - 1)

    plsc.subcore_barrier()

    for k in range(_RPT // _WB):
        r0 = s * _RPT + k * _WB
        pltpu.sync_copy(acc_sh.at[pl.ds(r0, _WB)],
                        out_hbm.at[c, pl.ds(r0, _WB)])

    @pl.when(s == _NS - 1)
    def _write_tail():
        pltpu.sync_copy(acc_sh.at[pl.ds(_RPT * _NS, _TAIL)],
                        out_hbm.at[c, pl.ds(_RPT * _NS, _TAIL)])


def _sc_edge(h, rc, s1, s2):
    mesh = plsc.VectorSubcoreMesh(core_axis_name="c", subcore_axis_name="s")
    cp = pltpu.CompilerParams()
    if "needs_layout_passes" in pltpu.CompilerParams.__dataclass_fields__:
        cp = dataclasses.replace(cp, needs_layout_passes=False)
    kfn = pl.kernel(
        _sc_edge_body,
        out_type=jax.ShapeDtypeStruct((_NC, _N, _H), jnp.float32),
        mesh=mesh,
        scratch_types=[
            pltpu.VMEM((_N,), jnp.float32),
            pltpu.VMEM((_N,), jnp.float32),
            pltpu.VMEM((4, 2, 2, _HC), jnp.int32),
            pltpu.VMEM((_HC,), jnp.float32),
            pltpu.VMEM((4, _HC, _H), jnp.float32),
            pltpu.VMEM_SHARED((_N, _H), jnp.float32),
            pltpu.SemaphoreType.DMA((4,)),
            pltpu.SemaphoreType.DMA((4,)),
            pltpu.SemaphoreType.DMA((4,)),
        ],
        compiler_params=cp,
    )
    return kfn(h, rc, s1, s2)


@jax.jit
def kernel(x, edge_index, W_in, b_in, W_att1, b_att1, eps1,
           W_att2, b_att2, eps2, W_cls, b_cls):
    ei = edge_index.astype(jnp.int32)
    rc = jnp.stack([ei[0].reshape(-1, 2, _HC), ei[1].reshape(-1, 2, _HC)],
                   axis=1)

    def att_pad(w_att, b_att):
        a = jnp.zeros((_H, 128), jnp.float32)
        a = a.at[:, 0].set(w_att[:_H, 0])
        a = a.at[:, 1].set(w_att[_H:, 0])
        v = jnp.zeros((1, 128), jnp.float32).at[0, 0].set(b_att[0])
        return a, v

    a1, v1 = att_pad(W_att1, b_att1)
    a2, v2 = att_pad(W_att2, b_att2)
    b_in2 = b_in.reshape(1, _H)
    bcls2 = b_cls.reshape(1, -1)
    e1 = jnp.broadcast_to(eps1, (1, _H)).astype(jnp.float32)
    e2 = jnp.broadcast_to(eps2, (1, _H)).astype(jnp.float32)

    h1, s = _tc_in(x, W_in, b_in2, a1, v1)
    p = _sc_edge(h1, rc, s[:, 0], s[:, 1])
    h2, s = _tc_combine(h1, p[0], p[1], e1, a2, v2)
    q = _sc_edge(h2, rc, s[:, 0], s[:, 1])
    return _tc_final(h2, q[0], q[1], e2, W_cls, bcls2)


# speedup vs baseline: 1.0013x; 1.0013x over previous
import dataclasses
import functools

import jax
import jax.numpy as jnp
from jax import lax
from jax.experimental import pallas as pl
from jax.experimental.pallas import tpu as pltpu
from jax.experimental.pallas import tpu_sc as plsc

_N = 10000
_E = 320000
_H = 128
_NC = 2
_NS = 16
_TILES = _NC * _NS
_EPT = _E // _TILES
_C = 80
_HC = _C // 2
_CHUNKS = _EPT // _C
_NHALF = 2 * _CHUNKS
_RPT = 624
_WB = 208
_TAIL = _N - _RPT * _NS


def _tc_in_body(x_ref, w_ref, b_ref, a_ref, v_ref, h_ref, s_ref):
    h = jnp.dot(x_ref[...], w_ref[...], preferred_element_type=jnp.float32) + b_ref[...]
    h = jnp.maximum(h, 0.0)
    h_ref[...] = h
    s_ref[...] = jnp.dot(h, a_ref[...], preferred_element_type=jnp.float32) + v_ref[...]


def _tc_combine_body(h_ref, p0_ref, p1_ref, e_ref, a_ref, v_ref,
                     h2_ref, s_ref):
    e = e_ref[...]
    agg = p0_ref[...] + p1_ref[...]
    h2 = jnp.maximum(e * h_ref[...] + (1.0 - e) * agg, 0.0)
    h2_ref[...] = h2
    s_ref[...] = jnp.dot(h2, a_ref[...], preferred_element_type=jnp.float32) + v_ref[...]


def _tc_final_body(h_ref, p0_ref, p1_ref, e_ref, w_ref, b_ref, o_ref):
    e = e_ref[...]
    agg = p0_ref[...] + p1_ref[...]
    h2 = jnp.maximum(e * h_ref[...] + (1.0 - e) * agg, 0.0)
    o_ref[...] = jnp.dot(h2, w_ref[...], preferred_element_type=jnp.float32) + b_ref[...]


def _tc_in(x, w, b, a, v):
    return pl.pallas_call(
        _tc_in_body,
        out_shape=(jax.ShapeDtypeStruct((_N, _H), jnp.float32),
                   jax.ShapeDtypeStruct((_N, _H), jnp.float32)),
    )(x, w, b, a, v)


def _tc_combine(h, p0, p1, e, a, v):
    return pl.pallas_call(
        _tc_combine_body,
        out_shape=(jax.ShapeDtypeStruct((_N, _H), jnp.float32),
                   jax.ShapeDtypeStruct((_N, _H), jnp.float32)),
    )(h, p0, p1, e, a, v)


def _tc_final(h, p0, p1, e, w, b):
    return pl.pallas_call(
        _tc_final_body,
        out_shape=jax.ShapeDtypeStruct((_N, b.shape[-1]), jnp.float32),
    )(h, p0, p1, e, w, b)


def _sc_edge_body(h_hbm, rc_hbm, s1_hbm, s2_hbm, out_hbm,
                  s1_v, s2_v, idx_ring, alpha_v, rows_ring,
                  acc_sh, isems, gsems, ssems):
    c = lax.axis_index("c")
    s = lax.axis_index("s")

    pltpu.sync_copy(s1_hbm, s1_v)
    pltpu.sync_copy(s2_hbm, s2_v)

    @pl.loop(0, _HC)
    def _zero_rows(i):
        for j in range(_H // 16):
            rows_ring[0, i, pl.ds(j * 16, 16)] = jnp.zeros((16,), jnp.float32)

    for k in range(15):
        pltpu.sync_copy(rows_ring.at[0],
                        acc_sh.at[pl.ds(s * _RPT + k * _HC, _HC)])
    pltpu.sync_copy(rows_ring.at[0, pl.ds(0, 24)],
                    acc_sh.at[pl.ds(s * _RPT + 600, 24)])

    @pl.when(s == _NS - 1)
    def _zero_tail():
        pltpu.sync_copy(rows_ring.at[0, pl.ds(0, _TAIL)],
                        acc_sh.at[pl.ds(_RPT * _NS, _TAIL)])

    plsc.subcore_barrier()

    cbase = (c * _NS + s) * _CHUNKS

    def idx_start(k):
        q = lax.rem(k, 4)
        pltpu.async_copy(rc_hbm.at[cbase + k], idx_ring.at[q], isems.at[q])

    def idx_wait(k):
        q = lax.rem(k, 4)
        pltpu.make_async_copy(rc_hbm.at[cbase], idx_ring.at[q],
                              isems.at[q]).wait()

    def _slots(j):
        q = lax.rem(lax.div(j, 2), 4)
        jh = lax.rem(j, 2)
        r = lax.rem(j, 4)
        return q, jh, r

    def gather_start(j):
        q, jh, r = _slots(j)
        pltpu.async_copy(h_hbm.at[idx_ring.at[q, 0, jh]], rows_ring.at[r],
                         gsems.at[r])

    def gather_wait(j):
        q, jh, r = _slots(j)
        pltpu.make_async_copy(h_hbm.at[idx_ring.at[q, 0, jh]],
                              rows_ring.at[r], gsems.at[r]).wait()

    def scatter_start(j):
        q, jh, r = _slots(j)
        pltpu.async_copy(rows_ring.at[r], acc_sh.at[idx_ring.at[q, 1, jh]],
                         ssems.at[r], add=True)

    def scatter_wait(j):
        q, jh, r = _slots(j)
        pltpu.make_async_copy(rows_ring.at[r],
                              acc_sh.at[idx_ring.at[q, 1, jh]],
                              ssems.at[r]).wait()

    def compute(j):
        q, jh, r = _slots(j)
        irf = idx_ring.at[q, 0, jh]
        icf = idx_ring.at[q, 1, jh]
        rv = rows_ring.at[r]

        for off in (0, 16, 24):
            ir = irf[pl.ds(off, 16)]
            ic = icf[pl.ds(off, 16)]
            z = plsc.load_gather(s1_v, [ir]) + plsc.load_gather(s2_v, [ic])
            t = jnp.exp(-2.0 * jnp.abs(z))
            m = (1.0 - t) / (1.0 + t)
            alpha_v[pl.ds(off, 16)] = jnp.sign(z) * m

        @plsc.parallel_loop(0, _HC, unroll=4)
        def _scale(e):
            av = plsc.load_gather(alpha_v, [jnp.broadcast_to(e, (16,))])
            for jj in range(_H // 16):
                sl = pl.ds(jj * 16, 16)
                rv[e, sl] = rv[e, sl] * av

    idx_start(0)
    idx_start(1)
    idx_start(2)
    idx_wait(0)
    gather_start(0)
    gather_start(1)

    @pl.loop(0, _CHUNKS)
    def _main(k):
        j = k * 2
        gather_wait(j)

        @pl.when(j >= 2)
        def _drain_scatter0():
            scatter_wait(j - 2)

        @pl.when(j + 2 < _NHALF)
        def _next_gather0():
            idx_wait(k + 1)
            gather_start(j + 2)

        compute(j)
        scatter_start(j)

        gather_wait(j + 1)

        @pl.when(j >= 1)
        def _drain_scatter1():
            scatter_wait(j - 1)

        @pl.when(j + 3 < _NHALF)
        def _next_gather1():
            gather_start(j + 3)

        @pl.when(k + 3 < _CHUNKS)
        def _prefetch_idx():
            idx_start(k + 3)

        compute(j + 1)
        scatter_start(j + 1)

    scatter_wait(_NHALF - 2)
    scatter_wait(_NHALF - 1)

    plsc.subcore_barrier()

    for k in range(_RPT // _WB):
        r0 = s * _RPT + k * _WB
        pltpu.sync_copy(acc_sh.at[pl.ds(r0, _WB)],
                        out_hbm.at[c, pl.ds(r0, _WB)])

    @pl.when(s == _NS - 1)
    def _write_tail():
        pltpu.sync_copy(acc_sh.at[pl.ds(_RPT * _NS, _TAIL)],
                        out_hbm.at[c, pl.ds(_RPT * _NS, _TAIL)])


def _sc_edge(h, rc, s1, s2):
    mesh = plsc.VectorSubcoreMesh(core_axis_name="c", subcore_axis_name="s")
    cp = pltpu.CompilerParams()
    if "needs_layout_passes" in pltpu.CompilerParams.__dataclass_fields__:
        cp = dataclasses.replace(cp, needs_layout_passes=False)
    kfn = pl.kernel(
        _sc_edge_body,
        out_type=jax.ShapeDtypeStruct((_NC, _N, _H), jnp.float32),
        mesh=mesh,
        scratch_types=[
            pltpu.VMEM((_N,), jnp.float32),
            pltpu.VMEM((_N,), jnp.float32),
            pltpu.VMEM((4, 2, 2, _HC), jnp.int32),
            pltpu.VMEM((_HC,), jnp.float32),
            pltpu.VMEM((4, _HC, _H), jnp.float32),
            pltpu.VMEM_SHARED((_N, _H), jnp.float32),
            pltpu.SemaphoreType.DMA((4,)),
            pltpu.SemaphoreType.DMA((4,)),
            pltpu.SemaphoreType.DMA((4,)),
        ],
        compiler_params=cp,
    )
    return kfn(h, rc, s1, s2)


@jax.jit
def kernel(x, edge_index, W_in, b_in, W_att1, b_att1, eps1,
           W_att2, b_att2, eps2, W_cls, b_cls):
    ei = edge_index.astype(jnp.int32)
    rc = jnp.stack([ei[0].reshape(-1, 2, _HC), ei[1].reshape(-1, 2, _HC)],
                   axis=1)

    def att_pad(w_att, b_att):
        a = jnp.zeros((_H, 128), jnp.float32)
        a = a.at[:, 0].set(w_att[:_H, 0])
        a = a.at[:, 1].set(w_att[_H:, 0])
        v = jnp.zeros((1, 128), jnp.float32).at[0, 0].set(b_att[0])
        return a, v

    a1, v1 = att_pad(W_att1, b_att1)
    a2, v2 = att_pad(W_att2, b_att2)
    b_in2 = b_in.reshape(1, _H)
    bcls2 = b_cls.reshape(1, -1)
    e1 = jnp.broadcast_to(eps1, (1, _H)).astype(jnp.float32)
    e2 = jnp.broadcast_to(eps2, (1, _H)).astype(jnp.float32)

    h1, s = _tc_in(x, W_in, b_in2, a1, v1)
    p = _sc_edge(h1, rc, s[:, 0], s[:, 1])
    h2, s = _tc_combine(h1, p[0], p[1], e1, a2, v2)
    q = _sc_edge(h2, rc, s[:, 0], s[:, 1])
    return _tc_final(h2, q[0], q[1], e2, W_cls, bcls2)
